# baseline (speedup 1.0000x reference)
import functools

import jax
import jax.numpy as jnp
from jax import lax
from jax.experimental import pallas as pl
from jax.experimental.pallas import tpu as pltpu
from jax.experimental.pallas import tpu_sc as plsc

DIM = 128
BATCH = 16384
NC, NS = 2, 16
NW = NC * NS
B_PER_W = BATCH // NW
CHUNK = 128
NCHUNK = B_PER_W // CHUNK

_mesh = plsc.VectorSubcoreMesh(core_axis_name="c", subcore_axis_name="s")


@functools.partial(
    pl.kernel,
    mesh=_mesh,
    out_type=jax.ShapeDtypeStruct((BATCH, DIM), jnp.float32),
    scratch_types=[
        pltpu.VMEM((NCHUNK, CHUNK), jnp.int32),
        pltpu.VMEM((B_PER_W, DIM), jnp.float32),
    ]
    + [pltpu.SemaphoreType.DMA] * NCHUNK
    + [pltpu.SemaphoreType.DMA],
)
def _gather_kernel(idx_hbm, table_hbm, out_hbm, idx_v, rows_v, *sems):
    gsems, ssem = sems[:NCHUNK], sems[NCHUNK]
    wid = lax.axis_index("s") * NC + lax.axis_index("c")
    base = wid * B_PER_W
    pltpu.sync_copy(idx_hbm.at[pl.ds(wid * NCHUNK, NCHUNK)], idx_v)
    gathers = [
        pltpu.async_copy(
            table_hbm.at[idx_v.at[j]],
            rows_v.at[pl.ds(j * CHUNK, CHUNK)],
            gsems[j],
        )
        for j in range(NCHUNK)
    ]
    stores = []
    for j in range(NCHUNK):
        gathers[j].wait()
        stores.append(
            pltpu.async_copy(
                rows_v.at[pl.ds(j * CHUNK, CHUNK)],
                out_hbm.at[pl.ds(base + j * CHUNK, CHUNK)],
                ssem,
            )
        )
    for s in stores:
        s.wait()


def kernel(timesteps, pe):
    idx2d = timesteps.astype(jnp.int32).reshape(NW * NCHUNK, CHUNK)
    return _gather_kernel(idx2d, pe)


# speedup vs baseline: 1.1701x; 1.1701x over previous
import functools

import jax
import jax.numpy as jnp
from jax import lax
from jax.experimental import pallas as pl
from jax.experimental.pallas import tpu as pltpu
from jax.experimental.pallas import tpu_sc as plsc

DIM = 128
ROWS = 1000
BATCH = 16384
NC, NS = 2, 16
NW = NC * NS
B_PER_W = BATCH // NW
CHUNK = 128
NCHUNK = B_PER_W // CHUNK
STAGE_WORKERS = 5
STAGE_ROWS = ROWS // STAGE_WORKERS

_mesh = plsc.VectorSubcoreMesh(core_axis_name="c", subcore_axis_name="s")


@functools.partial(
    pl.kernel,
    mesh=_mesh,
    out_type=jax.ShapeDtypeStruct((BATCH, DIM), jnp.float32),
    scratch_types=[
        pltpu.VMEM_SHARED((ROWS, DIM), jnp.float32),
        pltpu.VMEM((NCHUNK, CHUNK), jnp.int32),
        pltpu.VMEM((B_PER_W, DIM), jnp.float32),
    ]
    + [pltpu.SemaphoreType.DMA] * NCHUNK
    + [pltpu.SemaphoreType.DMA],
)
def _gather_kernel(idx_hbm, table_hbm, out_hbm, table_spm, idx_v, rows_v, *sems):
    gsems, ssem = sems[:NCHUNK], sems[NCHUNK]
    sid = lax.axis_index("s")
    wid = sid * NC + lax.axis_index("c")
    base = wid * B_PER_W

    @pl.when(sid < STAGE_WORKERS)
    def _stage():
        off = pl.multiple_of(sid * STAGE_ROWS, 8)
        pltpu.sync_copy(
            table_hbm.at[pl.ds(off, STAGE_ROWS)],
            table_spm.at[pl.ds(off, STAGE_ROWS)],
        )

    pltpu.sync_copy(idx_hbm.at[pl.ds(wid * NCHUNK, NCHUNK)], idx_v)
    plsc.subcore_barrier()
    gathers = [
        pltpu.async_copy(
            table_spm.at[idx_v.at[j]],
            rows_v.at[pl.ds(j * CHUNK, CHUNK)],
            gsems[j],
        )
        for j in range(NCHUNK)
    ]
    stores = []
    for j in range(NCHUNK):
        gathers[j].wait()
        stores.append(
            pltpu.async_copy(
                rows_v.at[pl.ds(j * CHUNK, CHUNK)],
                out_hbm.at[pl.ds(base + j * CHUNK, CHUNK)],
                ssem,
            )
        )
    for s in stores:
        s.wait()


def kernel(timesteps, pe):
    idx2d = timesteps.astype(jnp.int32).reshape(NW * NCHUNK, CHUNK)
    return _gather_kernel(idx2d, pe)
